# baseline (speedup 1.0000x reference)
import functools

import jax
import jax.numpy as jnp
from jax import lax
from jax.experimental import pallas as pl
from jax.experimental.pallas import tpu as pltpu
from jax.experimental.pallas import tpu_sc as plsc

_N = 10000
_E = 320000
_M = 3
_D_IN = 128
_H = 64
_D_OUT = 2
_WD = 5e-4

_NC = 2
_NS = 16
_NW = _NC * _NS
_SUB = 96
_NSUB = 3
_C = _SUB * _NSUB
_NCHUNK = 36
_EWP = _NCHUNK * _C
_AR = 10080
_DUMP = _N
_R = _M * _N
_SROW = _NCHUNK * _NSUB
_IDXR = 3 * _SROW
_ZBLK = 80


def _splat_lane(vec16, lane):
    idx = jnp.full((16, 1), lane, jnp.int32)
    return lax.gather(
        vec16, idx,
        lax.GatherDimensionNumbers(offset_dims=(), collapsed_slice_dims=(0,),
                                   start_index_map=(0,)),
        (1,), mode=lax.GatherScatterMode.PROMISE_IN_BOUNDS)


def _balanced(sid, nblocks):
    base = nblocks // _NS
    rem = nblocks % _NS
    cnt = base + jnp.where(sid < rem, 1, 0)
    start = sid * base + jnp.minimum(sid, rem)
    return start, cnt


def _sc_segsum_body(table_ref, gpk_ref, spk_ref, wpk_ref, out_ref,
                    acc, idxb, rows0, rows1, rows2,
                    isem, zsem, gsem0, gsem1, gsem2, ssem0, ssem1, ssem2):
    cid = lax.axis_index("c")
    sid = lax.axis_index("s")
    wid = sid * _NC + cid
    slot = ((rows0, gsem0, ssem0),
            (rows1, gsem1, ssem1),
            (rows2, gsem2, ssem2))

    zb_start, zb_cnt = _balanced(sid, _AR // _ZBLK)
    wb_start, wb_cnt = _balanced(sid, _N // _ZBLK)

    zero16 = jnp.zeros((16,), jnp.float32)

    def gather_issue(j, c):
        rows, gsem, _ = slot[j]
        for t in range(_NSUB):
            pltpu.async_copy(table_ref.at[idxb.at[c * _NSUB + t]],
                             rows.at[pl.ds(t * _SUB, _SUB)], gsem)

    def gather_wait(j, c):
        rows, gsem, _ = slot[j]
        for t in range(_NSUB):
            pltpu.make_async_copy(table_ref.at[idxb.at[c * _NSUB + t]],
                                  rows.at[pl.ds(t * _SUB, _SUB)], gsem).wait()

    def scatter_issue(j, c):
        rows, _, ssem = slot[j]
        for t in range(_NSUB):
            pltpu.async_copy(rows.at[pl.ds(t * _SUB, _SUB)],
                             acc.at[idxb.at[_SROW + c * _NSUB + t]],
                             ssem, add=True)

    def scatter_wait(j, c):
        rows, _, ssem = slot[j]
        for t in range(_NSUB):
            pltpu.make_async_copy(rows.at[pl.ds(t * _SUB, _SUB)],
                                  acc.at[idxb.at[_SROW + c * _NSUB + t]],
                                  ssem).wait()

    def multiply(j, c):
        rows, _, _ = slot[j]
        nb = _SUB // 16

        @pl.loop(0, _C // 16)
        def _mul(b):
            t = b // nb
            bb = b - t * nb
            wvec = lax.bitcast_convert_type(
                idxb[2 * _SROW + c * _NSUB + t, pl.ds(bb * 16, 16)],
                jnp.float32)
            r0 = b * 16
            nq = _H // 16
            for ep in range(8):
                e0, e1 = 2 * ep, 2 * ep + 1
                s0 = _splat_lane(wvec, e0)
                s1 = _splat_lane(wvec, e1)
                a = ([rows[r0 + e0, pl.ds(q * 16, 16)] for q in range(nq)]
                     + [rows[r0 + e1, pl.ds(q * 16, 16)] for q in range(nq)])
                prod = ([a[q] * s0 for q in range(nq)]
                        + [a[nq + q] * s1 for q in range(nq)])
                for q in range(nq):
                    rows[r0 + e0, pl.ds(q * 16, 16)] = prod[q]
                for q in range(nq):
                    rows[r0 + e1, pl.ds(q * 16, 16)] = prod[nq + q]

    @pl.loop(0, _M)
    def _path(p):
        row0 = ((p * _NW) + wid) * _SROW
        pltpu.async_copy(gpk_ref.at[pl.ds(row0, _SROW)],
                         idxb.at[pl.ds(0, _SROW)], isem)
        pltpu.async_copy(spk_ref.at[pl.ds(row0, _SROW)],
                         idxb.at[pl.ds(_SROW, _SROW)], isem)
        pltpu.async_copy(wpk_ref.at[pl.ds(row0, _SROW)],
                         idxb.at[pl.ds(2 * _SROW, _SROW)], isem)

        @pl.loop(0, _ZBLK)
        def _zero_stage(r):
            for q in range(_H // 16):
                rows0[r, pl.ds(q * 16, 16)] = zero16

        @pl.loop(0, zb_cnt)
        def _zero_issue(k):
            pltpu.async_copy(rows0.at[pl.ds(0, _ZBLK)],
                             acc.at[pl.ds((zb_start + k) * _ZBLK, _ZBLK)],
                             zsem)

        @pl.loop(0, zb_cnt)
        def _zero_drain(k):
            pltpu.make_async_copy(
                rows0.at[pl.ds(0, _ZBLK)],
                acc.at[pl.ds((zb_start + k) * _ZBLK, _ZBLK)], zsem).wait()

        for _d in range(3):
            pltpu.make_async_copy(gpk_ref.at[pl.ds(row0, _SROW)],
                                  idxb.at[pl.ds(0, _SROW)], isem).wait()
        plsc.subcore_barrier()

        gather_issue(0, 0)
        gather_issue(1, 1)

        @pl.loop(0, _NCHUNK // 3)
        def _ring(tt):
            c0 = 3 * tt
            for k in range(3):
                c = c0 + k
                j = k
                gather_wait(j, c)
                multiply(j, c)
                scatter_issue(j, c)
                j2 = (k + 2) % 3
                if k == 0:
                    @pl.when(tt > 0)
                    def _sw0(j2=j2, c=c):
                        scatter_wait(j2, c - 1)
                    gather_issue(j2, c + 2)
                else:
                    @pl.when(c + 2 < _NCHUNK)
                    def _swk(j2=j2, c=c):
                        scatter_wait(j2, c - 1)
                        gather_issue(j2, c + 2)

        for k in range(3):
            c = _NCHUNK - 3 + k
            scatter_wait(k % 3, c)

        plsc.subcore_barrier()

        @pl.loop(0, wb_cnt)
        def _wo_issue(k):
            r0 = (wb_start + k) * _ZBLK
            pltpu.async_copy(acc.at[pl.ds(r0, _ZBLK)],
                             out_ref.at[cid].at[pl.ds(p * _N + r0, _ZBLK)],
                             zsem)

        @pl.loop(0, wb_cnt)
        def _wo_drain(k):
            r0 = (wb_start + k) * _ZBLK
            pltpu.make_async_copy(
                acc.at[pl.ds(r0, _ZBLK)],
                out_ref.at[cid].at[pl.ds(p * _N + r0, _ZBLK)], zsem).wait()

        plsc.subcore_barrier()


def _sc_segsum(table, gpk, spk, wpk):
    mesh = plsc.VectorSubcoreMesh(core_axis_name="c", subcore_axis_name="s",
                                  num_cores=_NC, num_subcores=_NS)
    f = pl.kernel(
        _sc_segsum_body,
        out_type=jax.ShapeDtypeStruct((_NC, _R, _H), jnp.float32),
        mesh=mesh,
        scratch_types=[
            pltpu.VMEM_SHARED((_AR, _H), jnp.float32),
            pltpu.VMEM((_IDXR, _SUB), jnp.int32),
            pltpu.VMEM((_C, _H), jnp.float32),
            pltpu.VMEM((_C, _H), jnp.float32),
            pltpu.VMEM((_C, _H), jnp.float32),
            pltpu.SemaphoreType.DMA,
            pltpu.SemaphoreType.DMA,
            pltpu.SemaphoreType.DMA,
            pltpu.SemaphoreType.DMA,
            pltpu.SemaphoreType.DMA,
            pltpu.SemaphoreType.DMA,
            pltpu.SemaphoreType.DMA,
            pltpu.SemaphoreType.DMA,
        ],
        compiler_params=pltpu.CompilerParams(use_tc_tiling_on_sc=False),
    )
    return f(table, gpk, spk, wpk)


def _tc_matmul_body(x_ref, w_ref, o_ref):
    o_ref[...] = jnp.dot(x_ref[...], w_ref[...],
                         preferred_element_type=jnp.float32)


def _tc_matmul(x, w1):
    bm = 2000
    return pl.pallas_call(
        _tc_matmul_body,
        grid=(_N // bm,),
        in_specs=[
            pl.BlockSpec((bm, _D_IN), lambda i: (i, 0)),
            pl.BlockSpec((_D_IN, _H), lambda i: (0, 0)),
        ],
        out_specs=pl.BlockSpec((bm, _H), lambda i: (i, 0)),
        out_shape=jax.ShapeDtypeStruct((_N, _H), jnp.float32),
    )(x, w1)


def _tc_norm_body(p_ref, o_ref):
    h = p_ref[0] + p_ref[1]
    h = jnp.maximum(h, 0.0)
    s = jnp.sum(h * h, axis=1, keepdims=True)
    o_ref[...] = h * lax.rsqrt(jnp.maximum(s, 1e-12))


def _tc_norm(p):
    bm = 3000
    return pl.pallas_call(
        _tc_norm_body,
        grid=(_R // bm,),
        in_specs=[pl.BlockSpec((_NC, bm, _H), lambda i: (0, i, 0))],
        out_specs=pl.BlockSpec((bm, _H), lambda i: (i, 0)),
        out_shape=jax.ShapeDtypeStruct((_R, _H), jnp.float32),
    )(p)


_BE = 2000
_GE = _N // _BE


def _s2_block(pa, w2_ref):
    t = pa[0] + pa[1]
    return jnp.dot(t, w2_ref[...], preferred_element_type=jnp.float32)


def _tc_att_sums_body(p2a_ref, p2b_ref, p2c_ref, w2_ref, womA_ref, womB_ref,
                      msk_ref, sums_ref):
    i = pl.program_id(0)

    @pl.when(i == 0)
    def _init():
        for r in range(_M):
            for c in range(_M):
                sums_ref[r, c] = 0.0
        sums_ref[3, 0] = 0.0
        sums_ref[3, 1] = 0.0

    for p, pref in enumerate((p2a_ref, p2b_ref, p2c_ref)):
        s2 = _s2_block(pref[...], w2_ref)
        c0 = s2[:, 0:1]
        c1 = s2[:, 1:2]
        for j in range(_M):
            sums_ref[p, j] = sums_ref[p, j] + jnp.sum(
                c0 * womA_ref[:, j:j + 1]) + jnp.sum(c1 * womB_ref[:, j:j + 1])
    sums_ref[3, 0] = sums_ref[3, 0] + jnp.sum(msk_ref[...])
    sums_ref[3, 1] = (sums_ref[3, 1]
                      + jnp.sum(womA_ref[...] * womA_ref[...])
                      + jnp.sum(womB_ref[...] * womB_ref[...]))


def _tc_att_sums(p2, w2, womA, womB, msk):
    specs = [pl.BlockSpec((_NC, _BE, _H), functools.partial(
        lambda p, i: (0, p * _GE + i, 0), p)) for p in range(_M)]
    return pl.pallas_call(
        _tc_att_sums_body,
        grid=(_GE,),
        in_specs=specs + [
            pl.BlockSpec((_H, _D_OUT), lambda i: (0, 0)),
            pl.BlockSpec((_BE, _M), lambda i: (i, 0)),
            pl.BlockSpec((_BE, _M), lambda i: (i, 0)),
            pl.BlockSpec((_BE, 1), lambda i: (i, 0)),
        ],
        out_specs=pl.BlockSpec(memory_space=pltpu.SMEM),
        out_shape=jax.ShapeDtypeStruct((4, _M), jnp.float32),
    )(p2, p2, p2, w2, womA, womB, msk)


def _tc_loss_body(sums_ref, b_ref, u_ref, p2a_ref, p2b_ref, p2c_ref, w2_ref,
                  w1_ref, lbl_ref, msk_ref, loss_ref, acc_ref):
    i = pl.program_id(0)

    vu = []
    for p in range(_M):
        acc_p = 0.0
        for j in range(_M):
            vpj = jnp.tanh(sums_ref[p, j] + b_ref[0, j])
            acc_p = acc_p + vpj * u_ref[0, j]
        vu.append(acc_p)
    mx = jnp.maximum(jnp.maximum(vu[0], vu[1]), vu[2])
    e = [jnp.exp(v - mx) for v in vu]
    tot = e[0] + e[1] + e[2]
    alphas = [ek / tot for ek in e]

    att = 0.0
    for p, pref in enumerate((p2a_ref, p2b_ref, p2c_ref)):
        att = att + alphas[p] * _s2_block(pref[...], w2_ref)

    l0 = att[:, 0:1]
    l1 = att[:, 1:2]
    m = jnp.maximum(l0, l1)
    lse = m + jnp.log(jnp.exp(l0 - m) + jnp.exp(l1 - m))
    lbl = lbl_ref[...]
    sel = jnp.where(lbl == 0, l0, l1)
    ce = lse - sel

    mean_mask = sums_ref[3, 0] / _N
    mnorm = msk_ref[...] / mean_mask
    pred = jnp.where(l1 > l0, 1, 0)

    @pl.when(i == 0)
    def _init():
        bu_sq = 0.0
        for j in range(_M):
            bu_sq = bu_sq + b_ref[0, j] * b_ref[0, j] + u_ref[0, j] * u_ref[0, j]
        l2 = (jnp.sum(w1_ref[...] * w1_ref[...])
              + jnp.sum(w2_ref[...] * w2_ref[...])
              + sums_ref[3, 1] + bu_sq)
        loss_ref[0, 0] = _WD * 0.5 * l2
        acc_ref[0, 0] = 0.0

    loss_ref[0, 0] = loss_ref[0, 0] + jnp.sum(ce * mnorm) / _N
    acc_ref[0, 0] = acc_ref[0, 0] + jnp.sum(
        (pred == lbl).astype(jnp.float32) * mnorm) / _N


def _tc_loss(sums, b, u, p2, w2, w1, lbl, msk):
    pspecs = [pl.BlockSpec((_NC, _BE, _H), functools.partial(
        lambda p, i: (0, p * _GE + i, 0), p)) for p in range(_M)]
    return pl.pallas_call(
        _tc_loss_body,
        grid=(_GE,),
        in_specs=[
            pl.BlockSpec(memory_space=pltpu.SMEM),
            pl.BlockSpec(memory_space=pltpu.SMEM),
            pl.BlockSpec(memory_space=pltpu.SMEM),
        ] + pspecs + [
            pl.BlockSpec((_H, _D_OUT), lambda i: (0, 0)),
            pl.BlockSpec((_D_IN, _H), lambda i: (0, 0)),
            pl.BlockSpec((_BE, 1), lambda i: (i, 0)),
            pl.BlockSpec((_BE, 1), lambda i: (i, 0)),
        ],
        out_specs=(pl.BlockSpec(memory_space=pltpu.SMEM),
                   pl.BlockSpec(memory_space=pltpu.SMEM)),
        out_shape=(jax.ShapeDtypeStruct((1, 1), jnp.float32),
                   jax.ShapeDtypeStruct((1, 1), jnp.float32)),
    )(sums, b, u, p2, p2, p2, w2, w1, lbl, msk)


def kernel(x, edge_index, edge_weight, label, mask, W1, W2, w_omega,
           b_omega, u_omega):
    offs = (jnp.arange(_M, dtype=jnp.int32) * _N)[:, None]
    src = edge_index[:, 0, :]
    dst = edge_index[:, 1, :]
    wbits = lax.bitcast_convert_type(edge_weight, jnp.int32)
    pad_e = _NW * _EWP - _E
    sidx_p = jnp.pad(dst, ((0, 0), (0, pad_e)), constant_values=_DUMP)
    wbits_p = jnp.pad(wbits, ((0, 0), (0, pad_e)))

    def _rows(a):
        return a.reshape(-1, _SUB)

    gpk1 = _rows(jnp.pad(src, ((0, 0), (0, pad_e))))
    gpk2 = _rows(jnp.pad(src + offs, ((0, 0), (0, pad_e))))
    spk = _rows(sidx_p)
    wpk = _rows(wbits_p)
    wom3 = w_omega.reshape(_N, _D_OUT, _M)
    womA = wom3[:, 0, :]
    womB = wom3[:, 1, :]
    b2 = b_omega.reshape(1, _M)
    u2 = u_omega.reshape(1, _M)
    lbl2 = label.reshape(_N, 1)
    msk2 = mask.reshape(_N, 1)

    h0 = _tc_matmul(x, W1)
    p1 = _sc_segsum(h0, gpk1, spk, wpk)
    h = _tc_norm(p1)
    p2 = _sc_segsum(h, gpk2, spk, wpk)
    sums = _tc_att_sums(p2, W2, womA, womB, msk2)
    loss, acc = _tc_loss(sums, b2, u2, p2, W2, W1, lbl2, msk2)
    return loss.reshape(()), acc.reshape(())


# speedup vs baseline: 2.3600x; 2.3600x over previous
import functools

import jax
import jax.numpy as jnp
from jax import lax
from jax.experimental import pallas as pl
from jax.experimental.pallas import tpu as pltpu
from jax.experimental.pallas import tpu_sc as plsc

_N = 10000
_E = 320000
_M = 3
_D_IN = 128
_H = 64
_D_OUT = 2
_WD = 5e-4

_NC = 2
_NS = 16
_NW = _NC * _NS
_SUB = 112
_NSUB = 2
_C = _SUB * _NSUB
_NCHUNK = 45
_EWP = _NCHUNK * _C
_AR = _EWP
_DUMP = _N
_R = _M * _N
_SROW = _NCHUNK * _NSUB
_IDXR = 3 * _SROW
_ZBLK = 80


def _splat_lane(vec16, lane):
    idx = jnp.full((16, 1), lane, jnp.int32)
    return lax.gather(
        vec16, idx,
        lax.GatherDimensionNumbers(offset_dims=(), collapsed_slice_dims=(0,),
                                   start_index_map=(0,)),
        (1,), mode=lax.GatherScatterMode.PROMISE_IN_BOUNDS)


def _balanced(sid, nblocks):
    base = nblocks // _NS
    rem = nblocks % _NS
    cnt = base + jnp.where(sid < rem, 1, 0)
    start = sid * base + jnp.minimum(sid, rem)
    return start, cnt


def _sc_segsum_body(table_ref, gpk_ref, spk_ref, wpk_ref, out_ref,
                    acc, idxb, rows0, rows1, sbuf0, sbuf1,
                    isem, zsem, gsem0, gsem1, ssem0, ssem1):
    cid = lax.axis_index("c")
    sid = lax.axis_index("s")
    wid = sid * _NC + cid
    slot = ((rows0, sbuf0, gsem0, ssem0),
            (rows1, sbuf1, gsem1, ssem1))

    zb_start, zb_cnt = _balanced(sid, _AR // _ZBLK)
    wb_start, wb_cnt = _balanced(sid, _N // _ZBLK)

    zero16 = jnp.zeros((16,), jnp.float32)

    def gather_issue(j, c):
        rows, _, gsem, _ = slot[j]
        for t in range(_NSUB):
            pltpu.async_copy(table_ref.at[idxb.at[c * _NSUB + t]],
                             rows.at[pl.ds(t * _SUB, _SUB)], gsem)

    def slot_work(j, c, first):
        rows, sbuf, gsem, ssem = slot[j]
        if not first:
            for t in range(_NSUB):
                pltpu.make_async_copy(
                    sbuf.at[pl.ds(t * _SUB, _SUB)],
                    acc.at[idxb.at[_SROW + c * _NSUB + t]], ssem).wait()
        for t in range(_NSUB):
            pltpu.make_async_copy(table_ref.at[idxb.at[c * _NSUB + t]],
                                  rows.at[pl.ds(t * _SUB, _SUB)], gsem).wait()

        nb = _SUB // 16

        @pl.loop(0, _C // 16)
        def _mul(b):
            t = b // nb
            bb = b - t * nb
            wvec = lax.bitcast_convert_type(
                idxb[2 * _SROW + c * _NSUB + t, pl.ds(bb * 16, 16)],
                jnp.float32)
            r0 = b * 16
            nq = _H // 16
            for ep in range(8):
                e0, e1 = 2 * ep, 2 * ep + 1
                s0 = _splat_lane(wvec, e0)
                s1 = _splat_lane(wvec, e1)
                a = ([rows[r0 + e0, pl.ds(q * 16, 16)] for q in range(nq)]
                     + [rows[r0 + e1, pl.ds(q * 16, 16)] for q in range(nq)])
                prod = ([a[q] * s0 for q in range(nq)]
                        + [a[nq + q] * s1 for q in range(nq)])
                for q in range(nq):
                    sbuf[r0 + e0, pl.ds(q * 16, 16)] = prod[q]
                for q in range(nq):
                    sbuf[r0 + e1, pl.ds(q * 16, 16)] = prod[nq + q]

        for t in range(_NSUB):
            pltpu.async_copy(sbuf.at[pl.ds(t * _SUB, _SUB)],
                             acc.at[idxb.at[_SROW + c * _NSUB + t]],
                             ssem, add=True)

    @pl.loop(0, _M)
    def _path(p):
        row0 = ((p * _NW) + wid) * _SROW
        pltpu.async_copy(gpk_ref.at[pl.ds(row0, _SROW)],
                         idxb.at[pl.ds(0, _SROW)], isem)
        pltpu.async_copy(spk_ref.at[pl.ds(row0, _SROW)],
                         idxb.at[pl.ds(_SROW, _SROW)], isem)
        pltpu.async_copy(wpk_ref.at[pl.ds(row0, _SROW)],
                         idxb.at[pl.ds(2 * _SROW, _SROW)], isem)

        @pl.loop(0, _ZBLK)
        def _zero_stage(r):
            for q in range(_H // 16):
                sbuf0[r, pl.ds(q * 16, 16)] = zero16

        @pl.loop(0, zb_cnt)
        def _zero_issue(k):
            pltpu.async_copy(sbuf0.at[pl.ds(0, _ZBLK)],
                             acc.at[pl.ds((zb_start + k) * _ZBLK, _ZBLK)],
                             zsem)

        @pl.loop(0, zb_cnt)
        def _zero_drain(k):
            pltpu.make_async_copy(
                sbuf0.at[pl.ds(0, _ZBLK)],
                acc.at[pl.ds((zb_start + k) * _ZBLK, _ZBLK)], zsem).wait()

        for _d in range(3):
            pltpu.make_async_copy(gpk_ref.at[pl.ds(row0, _SROW)],
                                  idxb.at[pl.ds(0, _SROW)], isem).wait()
        plsc.subcore_barrier()

        gather_issue(0, 0)
        gather_issue(1, 1)
        slot_work(0, 0, first=True)
        gather_issue(0, 2)
        slot_work(1, 1, first=True)
        gather_issue(1, 3)

        @pl.loop(0, (_NCHUNK - 3) // 2)
        def _pair(tt):
            ca = 2 + 2 * tt
            for j in range(2):
                c = ca + j
                slot_work(j, c, first=False)

                @pl.when(c + 2 < _NCHUNK)
                def _sweep(j=j, c=c):
                    gather_issue(j, c + 2)

        slot_work(0, _NCHUNK - 1, first=False)
        for j, cl in ((0, _NCHUNK - 1), (1, _NCHUNK - 2)):
            _, sbuf, _, ssem = slot[j]
            for t in range(_NSUB):
                pltpu.make_async_copy(
                    sbuf.at[pl.ds(t * _SUB, _SUB)],
                    acc.at[idxb.at[_SROW + cl * _NSUB + t]], ssem).wait()

        plsc.subcore_barrier()

        @pl.loop(0, wb_cnt)
        def _wo_issue(k):
            r0 = (wb_start + k) * _ZBLK
            pltpu.async_copy(acc.at[pl.ds(r0, _ZBLK)],
                             out_ref.at[cid].at[pl.ds(p * _N + r0, _ZBLK)],
                             zsem)

        @pl.loop(0, wb_cnt)
        def _wo_drain(k):
            r0 = (wb_start + k) * _ZBLK
            pltpu.make_async_copy(
                acc.at[pl.ds(r0, _ZBLK)],
                out_ref.at[cid].at[pl.ds(p * _N + r0, _ZBLK)], zsem).wait()

        plsc.subcore_barrier()


def _sc_segsum(table, gpk, spk, wpk):
    mesh = plsc.VectorSubcoreMesh(core_axis_name="c", subcore_axis_name="s",
                                  num_cores=_NC, num_subcores=_NS)
    f = pl.kernel(
        _sc_segsum_body,
        out_type=jax.ShapeDtypeStruct((_NC, _R, _H), jnp.float32),
        mesh=mesh,
        scratch_types=[
            pltpu.VMEM_SHARED((_AR, _H), jnp.float32),
            pltpu.VMEM((_IDXR, _SUB), jnp.int32),
            pltpu.VMEM((_C, _H), jnp.float32),
            pltpu.VMEM((_C, _H), jnp.float32),
            pltpu.VMEM((_C, _H), jnp.float32),
            pltpu.VMEM((_C, _H), jnp.float32),
            pltpu.SemaphoreType.DMA,
            pltpu.SemaphoreType.DMA,
            pltpu.SemaphoreType.DMA,
            pltpu.SemaphoreType.DMA,
            pltpu.SemaphoreType.DMA,
            pltpu.SemaphoreType.DMA,
        ],
        compiler_params=pltpu.CompilerParams(use_tc_tiling_on_sc=False),
    )
    return f(table, gpk, spk, wpk)


def _tc_matmul_body(x_ref, w_ref, o_ref):
    o_ref[...] = jnp.dot(x_ref[...], w_ref[...],
                         preferred_element_type=jnp.float32)


def _tc_matmul(x, w1):
    bm = 2000
    return pl.pallas_call(
        _tc_matmul_body,
        grid=(_N // bm,),
        in_specs=[
            pl.BlockSpec((bm, _D_IN), lambda i: (i, 0)),
            pl.BlockSpec((_D_IN, _H), lambda i: (0, 0)),
        ],
        out_specs=pl.BlockSpec((bm, _H), lambda i: (i, 0)),
        out_shape=jax.ShapeDtypeStruct((_N, _H), jnp.float32),
    )(x, w1)


def _tc_norm_body(p_ref, o_ref):
    h = p_ref[0] + p_ref[1]
    h = jnp.maximum(h, 0.0)
    s = jnp.sum(h * h, axis=1, keepdims=True)
    o_ref[...] = h * lax.rsqrt(jnp.maximum(s, 1e-12))


def _tc_norm(p):
    bm = 3000
    return pl.pallas_call(
        _tc_norm_body,
        grid=(_R // bm,),
        in_specs=[pl.BlockSpec((_NC, bm, _H), lambda i: (0, i, 0))],
        out_specs=pl.BlockSpec((bm, _H), lambda i: (i, 0)),
        out_shape=jax.ShapeDtypeStruct((_R, _H), jnp.float32),
    )(p)


_BE = 2000
_GE = _N // _BE


def _s2_block(pa, w2_ref):
    t = pa[0] + pa[1]
    return jnp.dot(t, w2_ref[...], preferred_element_type=jnp.float32)


def _tc_att_sums_body(p2a_ref, p2b_ref, p2c_ref, w2_ref, womA_ref, womB_ref,
                      msk_ref, sums_ref):
    i = pl.program_id(0)

    @pl.when(i == 0)
    def _init():
        for r in range(_M):
            for c in range(_M):
                sums_ref[r, c] = 0.0
        sums_ref[3, 0] = 0.0
        sums_ref[3, 1] = 0.0

    for p, pref in enumerate((p2a_ref, p2b_ref, p2c_ref)):
        s2 = _s2_block(pref[...], w2_ref)
        c0 = s2[:, 0:1]
        c1 = s2[:, 1:2]
        for j in range(_M):
            sums_ref[p, j] = sums_ref[p, j] + jnp.sum(
                c0 * womA_ref[:, j:j + 1]) + jnp.sum(c1 * womB_ref[:, j:j + 1])
    sums_ref[3, 0] = sums_ref[3, 0] + jnp.sum(msk_ref[...])
    sums_ref[3, 1] = (sums_ref[3, 1]
                      + jnp.sum(womA_ref[...] * womA_ref[...])
                      + jnp.sum(womB_ref[...] * womB_ref[...]))


def _tc_att_sums(p2, w2, womA, womB, msk):
    specs = [pl.BlockSpec((_NC, _BE, _H), functools.partial(
        lambda p, i: (0, p * _GE + i, 0), p)) for p in range(_M)]
    return pl.pallas_call(
        _tc_att_sums_body,
        grid=(_GE,),
        in_specs=specs + [
            pl.BlockSpec((_H, _D_OUT), lambda i: (0, 0)),
            pl.BlockSpec((_BE, _M), lambda i: (i, 0)),
            pl.BlockSpec((_BE, _M), lambda i: (i, 0)),
            pl.BlockSpec((_BE, 1), lambda i: (i, 0)),
        ],
        out_specs=pl.BlockSpec(memory_space=pltpu.SMEM),
        out_shape=jax.ShapeDtypeStruct((4, _M), jnp.float32),
    )(p2, p2, p2, w2, womA, womB, msk)


def _tc_loss_body(sums_ref, b_ref, u_ref, p2a_ref, p2b_ref, p2c_ref, w2_ref,
                  w1_ref, lbl_ref, msk_ref, loss_ref, acc_ref):
    i = pl.program_id(0)

    vu = []
    for p in range(_M):
        acc_p = 0.0
        for j in range(_M):
            vpj = jnp.tanh(sums_ref[p, j] + b_ref[0, j])
            acc_p = acc_p + vpj * u_ref[0, j]
        vu.append(acc_p)
    mx = jnp.maximum(jnp.maximum(vu[0], vu[1]), vu[2])
    e = [jnp.exp(v - mx) for v in vu]
    tot = e[0] + e[1] + e[2]
    alphas = [ek / tot for ek in e]

    att = 0.0
    for p, pref in enumerate((p2a_ref, p2b_ref, p2c_ref)):
        att = att + alphas[p] * _s2_block(pref[...], w2_ref)

    l0 = att[:, 0:1]
    l1 = att[:, 1:2]
    m = jnp.maximum(l0, l1)
    lse = m + jnp.log(jnp.exp(l0 - m) + jnp.exp(l1 - m))
    lbl = lbl_ref[...]
    sel = jnp.where(lbl == 0, l0, l1)
    ce = lse - sel

    mean_mask = sums_ref[3, 0] / _N
    mnorm = msk_ref[...] / mean_mask
    pred = jnp.where(l1 > l0, 1, 0)

    @pl.when(i == 0)
    def _init():
        bu_sq = 0.0
        for j in range(_M):
            bu_sq = bu_sq + b_ref[0, j] * b_ref[0, j] + u_ref[0, j] * u_ref[0, j]
        l2 = (jnp.sum(w1_ref[...] * w1_ref[...])
              + jnp.sum(w2_ref[...] * w2_ref[...])
              + sums_ref[3, 1] + bu_sq)
        loss_ref[0, 0] = _WD * 0.5 * l2
        acc_ref[0, 0] = 0.0

    loss_ref[0, 0] = loss_ref[0, 0] + jnp.sum(ce * mnorm) / _N
    acc_ref[0, 0] = acc_ref[0, 0] + jnp.sum(
        (pred == lbl).astype(jnp.float32) * mnorm) / _N


def _tc_loss(sums, b, u, p2, w2, w1, lbl, msk):
    pspecs = [pl.BlockSpec((_NC, _BE, _H), functools.partial(
        lambda p, i: (0, p * _GE + i, 0), p)) for p in range(_M)]
    return pl.pallas_call(
        _tc_loss_body,
        grid=(_GE,),
        in_specs=[
            pl.BlockSpec(memory_space=pltpu.SMEM),
            pl.BlockSpec(memory_space=pltpu.SMEM),
            pl.BlockSpec(memory_space=pltpu.SMEM),
        ] + pspecs + [
            pl.BlockSpec((_H, _D_OUT), lambda i: (0, 0)),
            pl.BlockSpec((_D_IN, _H), lambda i: (0, 0)),
            pl.BlockSpec((_BE, 1), lambda i: (i, 0)),
            pl.BlockSpec((_BE, 1), lambda i: (i, 0)),
        ],
        out_specs=(pl.BlockSpec(memory_space=pltpu.SMEM),
                   pl.BlockSpec(memory_space=pltpu.SMEM)),
        out_shape=(jax.ShapeDtypeStruct((1, 1), jnp.float32),
                   jax.ShapeDtypeStruct((1, 1), jnp.float32)),
    )(sums, b, u, p2, p2, p2, w2, w1, lbl, msk)


def kernel(x, edge_index, edge_weight, label, mask, W1, W2, w_omega,
           b_omega, u_omega):
    offs = (jnp.arange(_M, dtype=jnp.int32) * _N)[:, None]
    src = edge_index[:, 0, :]
    dst = edge_index[:, 1, :]
    wbits = lax.bitcast_convert_type(edge_weight, jnp.int32)
    pad_e = _NW * _EWP - _E
    sidx_p = jnp.pad(dst, ((0, 0), (0, pad_e)), constant_values=_DUMP)
    wbits_p = jnp.pad(wbits, ((0, 0), (0, pad_e)))

    def _rows(a):
        return a.reshape(-1, _SUB)

    gpk1 = _rows(jnp.pad(src, ((0, 0), (0, pad_e))))
    gpk2 = _rows(jnp.pad(src + offs, ((0, 0), (0, pad_e))))
    spk = _rows(sidx_p)
    wpk = _rows(wbits_p)
    wom3 = w_omega.reshape(_N, _D_OUT, _M)
    womA = wom3[:, 0, :]
    womB = wom3[:, 1, :]
    b2 = b_omega.reshape(1, _M)
    u2 = u_omega.reshape(1, _M)
    lbl2 = label.reshape(_N, 1)
    msk2 = mask.reshape(_N, 1)

    h0 = _tc_matmul(x, W1)
    p1 = _sc_segsum(h0, gpk1, spk, wpk)
    h = _tc_norm(p1)
    p2 = _sc_segsum(h, gpk2, spk, wpk)
    sums = _tc_att_sums(p2, W2, womA, womB, msk2)
    loss, acc = _tc_loss(sums, b2, u2, p2, W2, W1, lbl2, msk2)
    return loss.reshape(()), acc.reshape(())
